# baseline (speedup 1.0000x reference)
import jax
import jax.numpy as jnp
from jax.experimental import pallas as pl
from jax.experimental.pallas import tpu as pltpu

N0, N3, D0, D3, H = 10000, 2000, 128, 2000, 64
R = 400
NSTEPS = N0 // R


def _t3_body(x3_ref, w3_ref, out_ref):
    out_ref[...] = jnp.dot(x3_ref[...].astype(jnp.bfloat16),
                           w3_ref[...].astype(jnp.bfloat16),
                           preferred_element_type=jnp.float32)


def _main_body(x0_ref, adj_ref, mask_ref, w0_ref, t3_ref, wp_ref,
               b64_ref, bp_ref, recon_hbm, h0_ref, h3t_ref,
               rbuf, sems):
    i = pl.program_id(0)

    def copy(step, slot):
        return pltpu.make_async_copy(
            rbuf.at[slot], recon_hbm.at[pl.ds(step * R, R)], sems.at[slot])

    slot = jax.lax.rem(i, 2)

    @pl.when(i >= 2)
    def _wait_prev():
        copy(i - 2, slot).wait()

    e = (mask_ref[...] * adj_ref[...]).astype(jnp.bfloat16)
    s0 = jnp.dot(x0_ref[...].astype(jnp.bfloat16),
                 w0_ref[...].astype(jnp.bfloat16),
                 preferred_element_type=jnp.float32)
    h0 = s0 + jnp.dot(e, t3_ref[...].astype(jnp.bfloat16),
                      preferred_element_type=jnp.float32) + b64_ref[...]
    h0_ref[...] = h0
    rbuf[slot] = jnp.dot(jnp.maximum(h0, 0.0).astype(jnp.bfloat16),
                         wp_ref[...].astype(jnp.bfloat16),
                         preferred_element_type=jnp.float32) + bp_ref[...]
    copy(i, slot).start()

    contrib = jax.lax.dot_general(
        s0.astype(jnp.bfloat16), e,
        dimension_numbers=(((0,), (0,)), ((), ())),
        preferred_element_type=jnp.float32)

    @pl.when(i == 0)
    def _init():
        h3t_ref[...] = contrib

    @pl.when(i > 0)
    def _acc():
        h3t_ref[...] += contrib

    @pl.when(i == NSTEPS - 1)
    def _drain():
        copy(i - 1, 1 - slot).wait()
        copy(i, slot).wait()


@jax.jit
def kernel(x0, x3, adj, mask, W0, b0, W3, b3, Wp, bp):
    t3 = pl.pallas_call(
        _t3_body,
        grid=(5,),
        in_specs=[
            pl.BlockSpec((N3 // 5, D3), lambda i: (i, 0)),
            pl.BlockSpec((D3, H), lambda i: (0, 0)),
        ],
        out_specs=pl.BlockSpec((N3 // 5, H), lambda i: (i, 0)),
        out_shape=jax.ShapeDtypeStruct((N3, H), jnp.float32),
    )(x3, W3)

    b64 = (b0 + b3).reshape(1, H)
    bp2 = bp.reshape(1, D3)

    recon, h0, h3t = pl.pallas_call(
        _main_body,
        grid=(NSTEPS,),
        in_specs=[
            pl.BlockSpec((R, D0), lambda i: (i, 0)),
            pl.BlockSpec((R, N3), lambda i: (i, 0)),
            pl.BlockSpec((R, N3), lambda i: (i, 0)),
            pl.BlockSpec((D0, H), lambda i: (0, 0)),
            pl.BlockSpec((N3, H), lambda i: (0, 0)),
            pl.BlockSpec((H, D3), lambda i: (0, 0)),
            pl.BlockSpec((1, H), lambda i: (0, 0)),
            pl.BlockSpec((1, D3), lambda i: (0, 0)),
        ],
        out_specs=[
            pl.BlockSpec(memory_space=pl.ANY),
            pl.BlockSpec((R, H), lambda i: (i, 0)),
            pl.BlockSpec((H, N3), lambda i: (0, 0)),
        ],
        out_shape=[
            jax.ShapeDtypeStruct((N0, D3), jnp.float32),
            jax.ShapeDtypeStruct((N0, H), jnp.float32),
            jax.ShapeDtypeStruct((H, N3), jnp.float32),
        ],
        scratch_shapes=[
            pltpu.VMEM((2, R, D3), jnp.float32),
            pltpu.SemaphoreType.DMA((2,)),
        ],
    )(x0, adj, mask, W0, t3, Wp, b64, bp2)

    h3 = h3t.T + t3 + (b0 + b3)
    return recon, h0, h3
